# Initial kernel scaffold; baseline (speedup 1.0000x reference)
import jax
import jax.numpy as jnp
from jax.experimental import pallas as pl


def kernel(entity, edge_index, edge_type, ent_table, proj_W, proj_b, rel_emb, W_l0, b_l0, W_r0, b_r0, W_e0, att0, bias0, W_l1, b_l1, W_r1, b_r1, W_e1, att1, bias1):
    raise NotImplementedError("write your pallas kernel here")


# speedup vs baseline: 6.5554x; 6.5554x over previous
import jax
import jax.numpy as jnp
import numpy as np
from jax.experimental import pallas as pl

N = 50000
E = 800000
D_IN = 128
D = 64
H = 4
C = 16
NEG = 0.2

_BN = 2000
_BE = 8000

_S = np.kron(np.eye(H, dtype=np.float32), np.ones((C, 1), np.float32))


def _mm_bias_k(x_ref, w_ref, b_ref, o_ref):
    o_ref[...] = (
        jnp.dot(x_ref[...], w_ref[...], preferred_element_type=jnp.float32)
        + b_ref[...]
    )


def _matmul_bias(x, w, b):
    n, k = x.shape
    m = w.shape[1]
    grid = pl.cdiv(n, _BN)
    return pl.pallas_call(
        _mm_bias_k,
        grid=(grid,),
        in_specs=[
            pl.BlockSpec((_BN, k), lambda i: (i, 0)),
            pl.BlockSpec((k, m), lambda i: (0, 0)),
            pl.BlockSpec((1, m), lambda i: (0, 0)),
        ],
        out_specs=pl.BlockSpec((_BN, m), lambda i: (i, 0)),
        out_shape=jax.ShapeDtypeStruct((n, m), jnp.float32),
    )(x, w, b.reshape(1, m))


def _edge_logits_k(xs_ref, xd_ref, ea_ref, att_ref, s_ref, o_ref):
    m = xs_ref[...] + xd_ref[...] + ea_ref[...]
    m = jnp.where(m > 0, m, NEG * m)
    w = m * att_ref[...]
    o_ref[...] = jnp.dot(w, s_ref[...], preferred_element_type=jnp.float32)


def _edge_logits(xs, xd, ea, att_flat, s):
    grid = pl.cdiv(E, _BE)
    return pl.pallas_call(
        _edge_logits_k,
        grid=(grid,),
        in_specs=[
            pl.BlockSpec((_BE, D), lambda i: (i, 0)),
            pl.BlockSpec((_BE, D), lambda i: (i, 0)),
            pl.BlockSpec((_BE, D), lambda i: (i, 0)),
            pl.BlockSpec((1, D), lambda i: (0, 0)),
            pl.BlockSpec((D, H), lambda i: (0, 0)),
        ],
        out_specs=pl.BlockSpec((_BE, H), lambda i: (i, 0)),
        out_shape=jax.ShapeDtypeStruct((E, H), jnp.float32),
    )(xs, xd, ea, att_flat, s)


def _edge_exp_k(lg_ref, mx_ref, o_ref):
    o_ref[...] = jnp.exp(lg_ref[...] - mx_ref[...])


def _edge_exp(logits, mx):
    grid = pl.cdiv(E, _BE)
    return pl.pallas_call(
        _edge_exp_k,
        grid=(grid,),
        in_specs=[
            pl.BlockSpec((_BE, H), lambda i: (i, 0)),
            pl.BlockSpec((_BE, H), lambda i: (i, 0)),
        ],
        out_specs=pl.BlockSpec((_BE, H), lambda i: (i, 0)),
        out_shape=jax.ShapeDtypeStruct((E, H), jnp.float32),
    )(logits, mx)


def _edge_msg_k(xs_ref, p_ref, dn_ref, st_ref, o_ref):
    alpha = p_ref[...] / (dn_ref[...] + 1e-16)
    a64 = jnp.dot(alpha, st_ref[...], preferred_element_type=jnp.float32)
    o_ref[...] = xs_ref[...] * a64


def _edge_msg(xs, p, dn, st):
    grid = pl.cdiv(E, _BE)
    return pl.pallas_call(
        _edge_msg_k,
        grid=(grid,),
        in_specs=[
            pl.BlockSpec((_BE, D), lambda i: (i, 0)),
            pl.BlockSpec((_BE, H), lambda i: (i, 0)),
            pl.BlockSpec((_BE, H), lambda i: (i, 0)),
            pl.BlockSpec((H, D), lambda i: (0, 0)),
        ],
        out_specs=pl.BlockSpec((_BE, D), lambda i: (i, 0)),
        out_shape=jax.ShapeDtypeStruct((E, D), jnp.float32),
    )(xs, p, dn, st)


def _bias_act_k(x_ref, b_ref, o_ref, *, elu):
    y = x_ref[...] + b_ref[...]
    if elu:
        y = jnp.where(y > 0, y, jnp.exp(jnp.minimum(y, 0.0)) - 1.0)
    o_ref[...] = y


def _bias_act(x, b, elu):
    import functools
    grid = pl.cdiv(N, _BN)
    return pl.pallas_call(
        functools.partial(_bias_act_k, elu=elu),
        grid=(grid,),
        in_specs=[
            pl.BlockSpec((_BN, D), lambda i: (i, 0)),
            pl.BlockSpec((1, D), lambda i: (0, 0)),
        ],
        out_specs=pl.BlockSpec((_BN, D), lambda i: (i, 0)),
        out_shape=jax.ShapeDtypeStruct((N, D), jnp.float32),
    )(x, b.reshape(1, D))


def _gat_layer(x, src, dst, edge_type, e_rel, W_l, b_l, W_r, b_r, att, bias,
               s_mat, st_mat, elu):
    x_l = _matmul_bias(x, W_l.T, b_l)
    x_r = _matmul_bias(x, W_r.T, b_r)
    xs = jnp.take(x_l, src, axis=0)
    xd = jnp.take(x_r, dst, axis=0)
    ea = jnp.take(e_rel, edge_type, axis=0)
    att_flat = att.reshape(1, D)
    logits = _edge_logits(xs, xd, ea, att_flat, s_mat)
    amax = jax.ops.segment_max(logits, dst, num_segments=N)
    p = _edge_exp(logits, jnp.take(amax, dst, axis=0))
    denom = jax.ops.segment_sum(p, dst, num_segments=N)
    msg = _edge_msg(xs, p, jnp.take(denom, dst, axis=0), st_mat)
    out = jax.ops.segment_sum(msg, dst, num_segments=N)
    return _bias_act(out, bias, elu)


def kernel(entity, edge_index, edge_type, ent_table, proj_W, proj_b, rel_emb,
           W_l0, b_l0, W_r0, b_r0, W_e0, att0, bias0,
           W_l1, b_l1, W_r1, b_r1, W_e1, att1, bias1):
    src = edge_index[0]
    dst = edge_index[1]
    s_mat = jnp.asarray(_S)
    st_mat = jnp.asarray(_S.T)

    x = jnp.take(ent_table, entity, axis=0)
    x = _matmul_bias(x, proj_W.T, proj_b)

    e_rel0 = rel_emb @ W_e0.T
    e_rel1 = rel_emb @ W_e1.T

    x = _gat_layer(x, src, dst, edge_type, e_rel0, W_l0, b_l0, W_r0, b_r0,
                   att0, bias0, s_mat, st_mat, elu=True)
    x = _gat_layer(x, src, dst, edge_type, e_rel1, W_l1, b_l1, W_r1, b_r1,
                   att1, bias1, s_mat, st_mat, elu=False)
    return x


# speedup vs baseline: 7.5607x; 1.1534x over previous
import jax
import jax.numpy as jnp
import numpy as np
from jax.experimental import pallas as pl

N = 50000
E = 800000
D_IN = 128
D = 64
H = 4
C = 16
NEG = 0.2

_BN = 2000
_BE = 8000

_S = np.kron(np.eye(H, dtype=np.float32), np.ones((C, 1), np.float32))


def _mm_bias_k(x_ref, w_ref, b_ref, o_ref):
    o_ref[...] = (
        jnp.dot(x_ref[...], w_ref[...], preferred_element_type=jnp.float32)
        + b_ref[...]
    )


def _matmul_bias(x, w, b):
    n, k = x.shape
    m = w.shape[1]
    grid = pl.cdiv(n, _BN)
    return pl.pallas_call(
        _mm_bias_k,
        grid=(grid,),
        in_specs=[
            pl.BlockSpec((_BN, k), lambda i: (i, 0)),
            pl.BlockSpec((k, m), lambda i: (0, 0)),
            pl.BlockSpec((1, m), lambda i: (0, 0)),
        ],
        out_specs=pl.BlockSpec((_BN, m), lambda i: (i, 0)),
        out_shape=jax.ShapeDtypeStruct((n, m), jnp.float32),
    )(x, w, b.reshape(1, m))


def _edge_logits_k(xs_ref, xd_ref, ea_ref, att_ref, s_ref, o_ref):
    m = xs_ref[...] + xd_ref[...] + ea_ref[...]
    m = jnp.where(m > 0, m, NEG * m)
    w = m * att_ref[...]
    o_ref[...] = jnp.dot(w, s_ref[...], preferred_element_type=jnp.float32)


def _edge_logits(xs, xd, ea, att_flat, s):
    grid = pl.cdiv(E, _BE)
    return pl.pallas_call(
        _edge_logits_k,
        grid=(grid,),
        in_specs=[
            pl.BlockSpec((_BE, D), lambda i: (i, 0)),
            pl.BlockSpec((_BE, D), lambda i: (i, 0)),
            pl.BlockSpec((_BE, D), lambda i: (i, 0)),
            pl.BlockSpec((1, D), lambda i: (0, 0)),
            pl.BlockSpec((D, H), lambda i: (0, 0)),
        ],
        out_specs=pl.BlockSpec((_BE, H), lambda i: (i, 0)),
        out_shape=jax.ShapeDtypeStruct((E, H), jnp.float32),
    )(xs, xd, ea, att_flat, s)


def _edge_pw_k(lg_ref, mx_ref, xs_ref, st_ref, p_ref, w_ref):
    p = jnp.exp(lg_ref[...] - mx_ref[...])
    p_ref[...] = p
    a64 = jnp.dot(p, st_ref[...], preferred_element_type=jnp.float32)
    w_ref[...] = xs_ref[...] * a64


def _edge_pw(logits, mx, xs, st):
    grid = pl.cdiv(E, _BE)
    return pl.pallas_call(
        _edge_pw_k,
        grid=(grid,),
        in_specs=[
            pl.BlockSpec((_BE, H), lambda i: (i, 0)),
            pl.BlockSpec((_BE, H), lambda i: (i, 0)),
            pl.BlockSpec((_BE, D), lambda i: (i, 0)),
            pl.BlockSpec((H, D), lambda i: (0, 0)),
        ],
        out_specs=[
            pl.BlockSpec((_BE, H), lambda i: (i, 0)),
            pl.BlockSpec((_BE, D), lambda i: (i, 0)),
        ],
        out_shape=[
            jax.ShapeDtypeStruct((E, H), jnp.float32),
            jax.ShapeDtypeStruct((E, D), jnp.float32),
        ],
    )(logits, mx, xs, st)


def _norm_bias_act_k(x_ref, dn_ref, st_ref, b_ref, o_ref, *, elu):
    d64 = jnp.dot(dn_ref[...], st_ref[...], preferred_element_type=jnp.float32)
    y = x_ref[...] / (d64 + 1e-16) + b_ref[...]
    if elu:
        y = jnp.where(y > 0, y, jnp.exp(jnp.minimum(y, 0.0)) - 1.0)
    o_ref[...] = y


def _norm_bias_act(x, dn, st, b, elu):
    import functools
    grid = pl.cdiv(N, _BN)
    return pl.pallas_call(
        functools.partial(_norm_bias_act_k, elu=elu),
        grid=(grid,),
        in_specs=[
            pl.BlockSpec((_BN, D), lambda i: (i, 0)),
            pl.BlockSpec((_BN, H), lambda i: (i, 0)),
            pl.BlockSpec((H, D), lambda i: (0, 0)),
            pl.BlockSpec((1, D), lambda i: (0, 0)),
        ],
        out_specs=pl.BlockSpec((_BN, D), lambda i: (i, 0)),
        out_shape=jax.ShapeDtypeStruct((N, D), jnp.float32),
    )(x, dn, st, b.reshape(1, D))


def _gat_layer(x, src, dst, edge_type, e_rel, W_l, b_l, W_r, b_r, att, bias,
               s_mat, st_mat, elu):
    x_l = _matmul_bias(x, W_l.T, b_l)
    x_r = _matmul_bias(x, W_r.T, b_r)
    xs = jnp.take(x_l, src, axis=0)
    xd = jnp.take(x_r, dst, axis=0)
    ea = jnp.take(e_rel, edge_type, axis=0)
    att_flat = att.reshape(1, D)
    logits = _edge_logits(xs, xd, ea, att_flat, s_mat)
    amax = jax.ops.segment_max(logits, dst, num_segments=N)
    p, w = _edge_pw(logits, jnp.take(amax, dst, axis=0), xs, st_mat)
    denom = jax.ops.segment_sum(p, dst, num_segments=N)
    num = jax.ops.segment_sum(w, dst, num_segments=N)
    return _norm_bias_act(num, denom, st_mat, bias, elu)


def kernel(entity, edge_index, edge_type, ent_table, proj_W, proj_b, rel_emb,
           W_l0, b_l0, W_r0, b_r0, W_e0, att0, bias0,
           W_l1, b_l1, W_r1, b_r1, W_e1, att1, bias1):
    src = edge_index[0]
    dst = edge_index[1]
    s_mat = jnp.asarray(_S)
    st_mat = jnp.asarray(_S.T)

    x = jnp.take(ent_table, entity, axis=0)
    x = _matmul_bias(x, proj_W.T, proj_b)

    e_rel0 = rel_emb @ W_e0.T
    e_rel1 = rel_emb @ W_e1.T

    x = _gat_layer(x, src, dst, edge_type, e_rel0, W_l0, b_l0, W_r0, b_r0,
                   att0, bias0, s_mat, st_mat, elu=True)
    x = _gat_layer(x, src, dst, edge_type, e_rel1, W_l1, b_l1, W_r1, b_r1,
                   att1, bias1, s_mat, st_mat, elu=False)
    return x
